# baseline (speedup 1.0000x reference)
import functools

import jax
import jax.numpy as jnp
from jax.experimental import pallas as pl

S = 2
BM = 512


def _matmul_kernel(x_ref, w0_ref, w1_ref, o_ref):
    x = x_ref[...]
    o_ref[0] = jnp.dot(w0_ref[...], x, preferred_element_type=jnp.float32)
    o_ref[1] = jnp.dot(w1_ref[...], x, preferred_element_type=jnp.float32)


@functools.partial(jax.jit, static_argnames=())
def kernel(input, weight):
    m, k = weight.shape
    _, n = input.shape
    half_tiles = m // S // BM
    out = pl.pallas_call(
        _matmul_kernel,
        grid=(half_tiles,),
        in_specs=[
            pl.BlockSpec((k, n), lambda i: (0, 0)),
            pl.BlockSpec((BM, k), lambda i: (i, 0)),
            pl.BlockSpec((BM, k), lambda i: (half_tiles + i, 0)),
        ],
        out_specs=pl.BlockSpec((S, BM, n), lambda i: (0, i, 0)),
        out_shape=jax.ShapeDtypeStruct((S, m // S, n), jnp.float32),
    )(input, weight, weight)
    return out.reshape(m, n)


# speedup vs baseline: 1.0297x; 1.0297x over previous
import functools

import jax
import jax.numpy as jnp
from jax.experimental import pallas as pl
from jax.experimental.pallas import tpu as pltpu

BM = 256
NBUF = 4


def _body(x_ref, w_ref, o_ref, *scratch):
    bufs = scratch[:NBUF]
    sems = scratch[NBUF:]
    m = w_ref.shape[0]
    nchunks = m // BM

    def start(i):
        pltpu.make_async_copy(
            w_ref.at[pl.ds(i * BM, BM), :], bufs[i % NBUF], sems[i % NBUF]
        ).start()

    for i in range(min(NBUF, nchunks)):
        start(i)
    x = x_ref[...]
    for i in range(nchunks):
        pltpu.make_async_copy(
            w_ref.at[pl.ds(i * BM, BM), :], bufs[i % NBUF], sems[i % NBUF]
        ).wait()
        o_ref[pl.ds(i * BM, BM), :] = jnp.dot(
            bufs[i % NBUF][...], x, preferred_element_type=jnp.float32
        )
        if i + NBUF < nchunks:
            start(i + NBUF)


@functools.partial(jax.jit, static_argnames=())
def kernel(input, weight):
    m, k = weight.shape
    _, n = input.shape
    return pl.pallas_call(
        _body,
        in_specs=[
            pl.BlockSpec(memory_space=pltpu.VMEM),
            pl.BlockSpec(memory_space=pltpu.MemorySpace.HBM),
        ],
        out_specs=pl.BlockSpec(memory_space=pltpu.VMEM),
        out_shape=jax.ShapeDtypeStruct((m, n), jnp.float32),
        scratch_shapes=(
            [pltpu.VMEM((BM, k), jnp.float32) for _ in range(NBUF)]
            + [pltpu.SemaphoreType.DMA for _ in range(NBUF)]
        ),
    )(input, weight)
